# speedup vs baseline: 1.0994x; 1.0994x over previous
import jax
import jax.numpy as jnp
from jax.experimental import pallas as pl

N = 10000
F = 128
EPS = 0.1
BLK = 1000


def _final_body(p_ref, x_ref, c0_ref, W_ref, b_ref, o_ref):
    out = p_ref[...] + x_ref[...] * c0_ref[...]
    o_ref[...] = jnp.dot(out, W_ref[...], preferred_element_type=jnp.float32) + b_ref[...]


def kernel(x, edge_index, att_l, att_r, W, b):
    n = x.shape[0]
    src = edge_index[0]
    dst = edge_index[1]
    deg = jnp.ones((n,), x.dtype).at[dst].add(1.0)
    dinv = jax.lax.rsqrt(deg)
    al = x @ att_l
    ar = x @ att_r
    c0 = EPS + jnp.tanh(al + ar) * dinv * dinv
    ew = dinv[src] * dinv[dst]
    alpha = jnp.tanh(al[src] + ar[dst])
    msg = x[src] * (alpha * ew)[:, None]
    p = jnp.zeros_like(x).at[dst].add(msg)

    grid = (n // BLK,)
    return pl.pallas_call(
        _final_body,
        grid=grid,
        in_specs=[
            pl.BlockSpec((BLK, F), lambda i: (i, 0)),
            pl.BlockSpec((BLK, F), lambda i: (i, 0)),
            pl.BlockSpec((BLK, 1), lambda i: (i, 0)),
            pl.BlockSpec((F, F), lambda i: (0, 0)),
            pl.BlockSpec((F,), lambda i: (0,)),
        ],
        out_specs=pl.BlockSpec((BLK, F), lambda i: (i, 0)),
        out_shape=jax.ShapeDtypeStruct((n, F), x.dtype),
    )(p, x, c0[:, None], W, b)


# speedup vs baseline: 17.9998x; 16.3729x over previous
import dataclasses

import jax
import jax.numpy as jnp
from jax import lax
from jax.experimental import pallas as pl
from jax.experimental.pallas import tpu as pltpu
from jax.experimental.pallas import tpu_sc as plsc

N = 10000
F = 128
EPS = 0.1
L = 16
NP = 10240
ROWS_PT = NP // 16
E = 320000
CHUNK = 128
NCH = 79
EPT = NCH * CHUNK
EPAD = 32 * EPT
BLK = 1000

_MESH = plsc.VectorSubcoreMesh(core_axis_name="c", subcore_axis_name="s")

_SC_PARAMS = pltpu.CompilerParams()
if "needs_layout_passes" in pltpu.CompilerParams.__dataclass_fields__:
    _SC_PARAMS = dataclasses.replace(_SC_PARAMS, needs_layout_passes=False)


def _deg_body(dst_hbm, deg_hbm, hist, idx_v):
    c = lax.axis_index("c")
    s = lax.axis_index("s")
    w = c * 16 + s

    @pl.loop(0, NP // L)
    def _(i):
        hist[pl.ds(i * L, L)] = jnp.zeros((L,), jnp.float32)

    base = w * EPT

    @pl.loop(0, NCH)
    def _(g):
        pltpu.sync_copy(dst_hbm.at[pl.ds(base + g * CHUNK, CHUNK)], idx_v)
        for v in range(0, CHUNK, L):
            iv = idx_v[pl.ds(v, L)]
            plsc.addupdate_scatter(hist, [iv], jnp.ones((L,), jnp.float32))

    pltpu.sync_copy(hist, deg_hbm.at[w])


def _node_body(x_ref, attl_ref, attr_ref, dp_ref, al_ref, ar_ref, di_ref):
    xv = x_ref[...]
    al_ref[...] = jnp.sum(xv * attl_ref[...], axis=1, keepdims=True)
    ar_ref[...] = jnp.sum(xv * attr_ref[...], axis=1, keepdims=True)
    deg = jnp.sum(dp_ref[...], axis=0, keepdims=True) + 1.0
    di = lax.rsqrt(deg)
    colid = lax.broadcasted_iota(jnp.int32, (1, NP), 1)
    di_ref[...] = jnp.where(colid < N, di, 0.0)


def _edge_body(x_hbm, src_hbm, dst_hbm, al_hbm, ar_hbm, di_hbm, out_hbm,
               acc, al_v, ar_v, di_v, sidx, didx, rows, coef, sem):
    c = lax.axis_index("c")
    s = lax.axis_index("s")

    pltpu.sync_copy(al_hbm, al_v)
    pltpu.sync_copy(ar_hbm, ar_v)
    pltpu.sync_copy(di_hbm, di_v)

    @pl.loop(0, CHUNK)
    def _(i):
        for j in range(0, F, L):
            rows[i, pl.ds(j, L)] = jnp.zeros((L,), jnp.float32)

    for k in range(ROWS_PT // CHUNK):
        pltpu.sync_copy(rows, acc.at[pl.ds(s * ROWS_PT + k * CHUNK, CHUNK)])
    plsc.subcore_barrier()

    base = (c * 16 + s) * EPT

    @pl.loop(0, NCH)
    def _(g):
        eb = base + g * CHUNK
        pltpu.sync_copy(src_hbm.at[pl.ds(eb, CHUNK)], sidx)
        pltpu.sync_copy(dst_hbm.at[pl.ds(eb, CHUNK)], didx)
        pltpu.async_copy(x_hbm.at[sidx], rows, sem).wait()

        for v in range(0, CHUNK, L):
            sv = sidx[pl.ds(v, L)]
            dv = didx[pl.ds(v, L)]
            a = plsc.load_gather(al_v, [sv])
            b2 = plsc.load_gather(ar_v, [dv])
            es = plsc.load_gather(di_v, [sv])
            ed = plsc.load_gather(di_v, [dv])
            z = a + b2
            e2 = jnp.exp(jnp.abs(z) * 2.0)
            t = jnp.sign(z) * (1.0 - 2.0 / (e2 + 1.0))
            coef[pl.ds(v, L)] = t * es * ed

        @pl.loop(0, CHUNK)
        def _(r):
            cv = plsc.load_gather(coef, [jnp.full((L,), r, jnp.int32)])
            for j in range(0, F, L):
                rows[r, pl.ds(j, L)] = rows[r, pl.ds(j, L)] * cv

        pltpu.sync_copy(rows, acc.at[didx], add=True)

    plsc.subcore_barrier()
    pltpu.sync_copy(acc.at[pl.ds(s * ROWS_PT, ROWS_PT)],
                    out_hbm.at[c, pl.ds(s * ROWS_PT, ROWS_PT)])


def _final_body(p0_ref, p1_ref, x_ref, al_ref, ar_ref, di_ref, W_ref, b_ref, o_ref):
    di = di_ref[...]
    c0 = EPS + jnp.tanh(al_ref[...] + ar_ref[...]) * di * di
    acc = p0_ref[...] + p1_ref[...] + x_ref[...] * c0
    o_ref[...] = jnp.dot(acc, W_ref[...], preferred_element_type=jnp.float32) + b_ref[...]


def kernel(x, edge_index, att_l, att_r, W, b):
    src = edge_index[0]
    dst = edge_index[1]
    pad_e = EPAD - E
    srcp = jnp.concatenate([src, jnp.full((pad_e,), N, jnp.int32)])
    dstp = jnp.concatenate([dst, jnp.full((pad_e,), N, jnp.int32)])
    xp = jnp.pad(x, ((0, NP - N), (0, 0)))

    deg_parts = pl.kernel(
        _deg_body,
        out_type=jax.ShapeDtypeStruct((32, NP), jnp.float32),
        mesh=_MESH,
        compiler_params=_SC_PARAMS,
        scratch_types=[
            pltpu.VMEM((NP,), jnp.float32),
            pltpu.VMEM((CHUNK,), jnp.int32),
        ],
    )(dstp)

    al2, ar2, di2 = pl.pallas_call(
        _node_body,
        out_shape=[
            jax.ShapeDtypeStruct((NP, 1), jnp.float32),
            jax.ShapeDtypeStruct((NP, 1), jnp.float32),
            jax.ShapeDtypeStruct((1, NP), jnp.float32),
        ],
    )(xp, att_l[None, :], att_r[None, :], deg_parts)

    parts = pl.kernel(
        _edge_body,
        out_type=jax.ShapeDtypeStruct((2, NP, F), jnp.float32),
        mesh=_MESH,
        compiler_params=_SC_PARAMS,
        scratch_types=[
            pltpu.VMEM_SHARED((NP, F), jnp.float32),
            pltpu.VMEM((NP,), jnp.float32),
            pltpu.VMEM((NP,), jnp.float32),
            pltpu.VMEM((NP,), jnp.float32),
            pltpu.VMEM((CHUNK,), jnp.int32),
            pltpu.VMEM((CHUNK,), jnp.int32),
            pltpu.VMEM((CHUNK, F), jnp.float32),
            pltpu.VMEM((CHUNK,), jnp.float32),
            pltpu.SemaphoreType.DMA,
        ],
    )(xp, srcp, dstp, al2.reshape(NP), ar2.reshape(NP), di2.reshape(NP))

    out = pl.pallas_call(
        _final_body,
        grid=(N // BLK,),
        in_specs=[
            pl.BlockSpec((BLK, F), lambda i: (i, 0)),
            pl.BlockSpec((BLK, F), lambda i: (i, 0)),
            pl.BlockSpec((BLK, F), lambda i: (i, 0)),
            pl.BlockSpec((BLK, 1), lambda i: (i, 0)),
            pl.BlockSpec((BLK, 1), lambda i: (i, 0)),
            pl.BlockSpec((BLK, 1), lambda i: (i, 0)),
            pl.BlockSpec((F, F), lambda i: (0, 0)),
            pl.BlockSpec((1, F), lambda i: (0, 0)),
        ],
        out_specs=pl.BlockSpec((BLK, F), lambda i: (i, 0)),
        out_shape=jax.ShapeDtypeStruct((N, F), jnp.float32),
    )(parts[0, :N], parts[1, :N], x, al2[:N], ar2[:N],
      di2.reshape(NP, 1)[:N], W, b[None, :])
    return out
